# baseline (speedup 1.0000x reference)
import jax
import jax.numpy as jnp
from jax.experimental import pallas as pl
from jax.experimental.pallas import tpu as pltpu

NUM_HEADS = 12
SEQ = 256
HEAD_DIM = 64


def _copy_kernel(rev_ref, out_ref, sems):
    i = pl.program_id(0)

    def _copy(row):
        return pltpu.make_async_copy(
            rev_ref.at[:, pl.ds(SEQ - row, SEQ), :],
            out_ref.at[:, row],
            sems.at[row % 2],
        )

    _copy(i).start()

    @pl.when(i > 0)
    def _():
        _copy(i - 1).wait()

    @pl.when(i == SEQ - 1)
    def _():
        _copy(i).wait()


def kernel(seq_length, relative_positional_encoding):
    start = seq_length - SEQ
    sl = jax.lax.dynamic_slice(
        relative_positional_encoding,
        (0, start, 0),
        (NUM_HEADS, 2 * SEQ, HEAD_DIM),
    )
    rev = sl[:, ::-1, :]

    return pl.pallas_call(
        _copy_kernel,
        grid=(SEQ,),
        in_specs=[
            pl.BlockSpec((NUM_HEADS, 2 * SEQ, HEAD_DIM), lambda i: (0, 0, 0)),
        ],
        out_specs=pl.BlockSpec(memory_space=pltpu.MemorySpace.HBM),
        out_shape=jax.ShapeDtypeStruct(
            (NUM_HEADS, SEQ, SEQ, HEAD_DIM), jnp.float32
        ),
        scratch_shapes=[pltpu.SemaphoreType.DMA((2,))],
    )(rev)


# speedup vs baseline: 1.0618x; 1.0618x over previous
import jax
import jax.numpy as jnp
from jax.experimental import pallas as pl
from jax.experimental.pallas import tpu as pltpu

NUM_HEADS = 12
SEQ = 256
HEAD_DIM = 64
NSEM = 8


def _copy_kernel(rev_ref, out_ref, sems):
    i = pl.program_id(0)

    def _copy(row):
        return pltpu.make_async_copy(
            rev_ref.at[:, pl.ds(SEQ - row, SEQ), :],
            out_ref.at[:, row],
            sems.at[row % NSEM],
        )

    _copy(i).start()

    @pl.when(i >= NSEM - 1)
    def _():
        _copy(i - (NSEM - 1)).wait()

    @pl.when(i == SEQ - 1)
    def _():
        for k in range(NSEM - 2, -1, -1):
            _copy(SEQ - 1 - k).wait()


def kernel(seq_length, relative_positional_encoding):
    start = seq_length - SEQ
    sl = jax.lax.dynamic_slice(
        relative_positional_encoding,
        (0, start, 0),
        (NUM_HEADS, 2 * SEQ, HEAD_DIM),
    )
    rev = sl[:, ::-1, :]

    return pl.pallas_call(
        _copy_kernel,
        grid=(SEQ,),
        in_specs=[
            pl.BlockSpec((NUM_HEADS, 2 * SEQ, HEAD_DIM), lambda i: (0, 0, 0)),
        ],
        out_specs=pl.BlockSpec(memory_space=pltpu.MemorySpace.HBM),
        out_shape=jax.ShapeDtypeStruct(
            (NUM_HEADS, SEQ, SEQ, HEAD_DIM), jnp.float32
        ),
        scratch_shapes=[pltpu.SemaphoreType.DMA((NSEM,))],
    )(rev)
